# baseline (speedup 1.0000x reference)
import functools

import jax
import jax.numpy as jnp
from jax import lax
from jax.experimental import pallas as pl
from jax.experimental.pallas import tpu as pltpu
from jax.experimental.pallas import tpu_sc as plsc

G = 256
NPTS = 4096 * 512
NC, NS, L = 2, 16, 16
NW = NC * NS
PTS_PER_W = NPTS // NW
P = 4096
N_CHUNKS = PTS_PER_W // P


@functools.partial(
    pl.kernel,
    out_type=jax.ShapeDtypeStruct((NPTS,), jnp.int32),
    mesh=plsc.VectorSubcoreMesh(core_axis_name="c", subcore_axis_name="s"),
    compiler_params=pltpu.CompilerParams(needs_layout_passes=False),
    scratch_types=[
        pltpu.VMEM((3 * P,), jnp.float32),
        pltpu.VMEM((P,), jnp.int32),
        pltpu.VMEM((P,), jnp.int32),
        pltpu.VMEM((P,), jnp.int32),
        pltpu.VMEM((P,), jnp.int32),
        pltpu.SemaphoreType.DMA,
    ],
)
def _sc_gather(pos_hbm, tbl_hbm, out_hbm, posv, linv, widxv, wordsv, outv, sem):
    wid = lax.axis_index("s") * NC + lax.axis_index("c")
    i3 = lax.iota(jnp.int32, L) * 3

    def compute_indices(g, carry):
        b = g * (3 * L)
        gx = plsc.load_gather(posv, [i3 + b])
        gy = plsc.load_gather(posv, [i3 + (b + 1)])
        gz = plsc.load_gather(posv, [i3 + (b + 2)])
        ix = (gx * float(G)).astype(jnp.int32)
        iy = (gy * float(G)).astype(jnp.int32)
        iz = (gz * float(G)).astype(jnp.int32)
        lin = ix * (G * G) + iy * G + iz
        s = pl.ds(g * L, L)
        linv[s] = lin
        widxv[s] = lax.shift_right_logical(lin, 2)
        return carry

    def extract_bytes(g, carry):
        s = pl.ds(g * L, L)
        w = wordsv[s]
        byte_sel = (linv[s] & 3) * 8
        outv[s] = lax.shift_right_logical(w, byte_sel) & 255
        return carry

    for it in range(N_CHUNKS):
        base = wid * PTS_PER_W + it * P
        pltpu.sync_copy(pos_hbm.at[pl.ds(base * 3, 3 * P)], posv)
        lax.fori_loop(0, P // L, compute_indices, 0)
        pltpu.async_copy(tbl_hbm.at[widxv], wordsv, sem).wait()
        lax.fori_loop(0, P // L, extract_bytes, 0)
        pltpu.sync_copy(outv, out_hbm.at[pl.ds(base, P)])


def kernel(positions, is_training, voxel_grid):
    pos_flat = positions.reshape(-1)
    tbl = lax.bitcast_convert_type(
        voxel_grid.reshape(-1, 4).astype(jnp.uint8), jnp.int32
    )
    out = _sc_gather(pos_flat, tbl)
    return out.reshape(positions.shape[:-1]).astype(jnp.bool_)


# speedup vs baseline: 1.1308x; 1.1308x over previous
import functools

import jax
import jax.numpy as jnp
from jax import lax
from jax.experimental import pallas as pl
from jax.experimental.pallas import tpu as pltpu
from jax.experimental.pallas import tpu_sc as plsc

G = 256
NROWS, NCOLS = 4096, 512
NC, NS, L = 2, 16, 16
NW = NC * NS
ROWS_PER_W = NROWS // NW
ROWS_PER_CHUNK = 8
P = ROWS_PER_CHUNK * NCOLS
N_CHUNKS = ROWS_PER_W // ROWS_PER_CHUNK
PLANES_PER_TILE = G // NS
WORDS_PER_PLANE = G * G // 32


@functools.partial(
    pl.kernel,
    out_type=jax.ShapeDtypeStruct((NROWS, NCOLS), jnp.int32),
    mesh=plsc.VectorSubcoreMesh(core_axis_name="c", subcore_axis_name="s"),
    compiler_params=pltpu.CompilerParams(
        needs_layout_passes=False, use_tc_tiling_on_sc=False
    ),
    scratch_types=[
        pltpu.VMEM_SHARED((G * G * G // 32,), jnp.int32),
        pltpu.VMEM((G, G), jnp.uint8),
        pltpu.VMEM((WORDS_PER_PLANE,), jnp.int32),
        pltpu.VMEM((ROWS_PER_CHUNK, NCOLS, 3), jnp.float32),
        pltpu.VMEM((P,), jnp.int32),
        pltpu.VMEM((P,), jnp.int32),
        pltpu.VMEM((P,), jnp.int32),
        pltpu.VMEM((ROWS_PER_CHUNK, NCOLS), jnp.int32),
        pltpu.SemaphoreType.DMA,
    ],
)
def _sc_prune(pos_hbm, grid_hbm, out_hbm, tbl_sp, planev, wordbuf, posv, linv,
              widxv, wordsv, outv, sem):
    cid = lax.axis_index("c")
    sid = lax.axis_index("s")
    wid = sid * NC + cid
    iota = lax.iota(jnp.int32, L)
    c0 = jnp.zeros((L,), jnp.int32)
    c1 = jnp.full((L,), 1, jnp.int32)
    c2 = jnp.full((L,), 2, jnp.int32)

    def pack_vec(ov, carry):
        r = ov * 2
        w = None
        for j in range(8):
            v8 = planev[r + (j >> 2), pl.ds((j & 3) * 64, 64)]
            v = plsc.bitcast(v8, jnp.int32)
            nib = lax.shift_right_logical(v * 0x08040201, 24)
            w = nib if j == 0 else w | (nib << (4 * j))
        wordbuf[pl.ds(ov * L, L)] = w
        return carry

    for pi in range(PLANES_PER_TILE):
        p = sid * PLANES_PER_TILE + pi
        pltpu.sync_copy(grid_hbm.at[p], planev)
        lax.fori_loop(0, WORDS_PER_PLANE // L, pack_vec, 0)
        pltpu.sync_copy(wordbuf, tbl_sp.at[pl.ds(p * WORDS_PER_PLANE,
                                                 WORDS_PER_PLANE)])
    plsc.subcore_barrier()

    def compute_indices(g, carry):
        rowv = jnp.full((L,), 0, jnp.int32) + lax.shift_right_logical(g, 5)
        colv = iota + ((g & 31) * L)
        gx = plsc.load_gather(posv, [rowv, colv, c0])
        gy = plsc.load_gather(posv, [rowv, colv, c1])
        gz = plsc.load_gather(posv, [rowv, colv, c2])
        ix = (gx * float(G)).astype(jnp.int32)
        iy = (gy * float(G)).astype(jnp.int32)
        iz = (gz * float(G)).astype(jnp.int32)
        lin = (ix << 16) | (iy << 8) | iz
        s = pl.ds(g * L, L)
        linv[s] = lin
        widxv[s] = (lax.shift_right_logical(lin, 5) & -16) | (
            lax.shift_right_logical(lin, 2) & 15)
        return carry

    def extract_bits(g, carry):
        s = pl.ds(g * L, L)
        lin = linv[s]
        b = (lax.shift_right_logical(lin, 4) & 28) | ((lin & 3) ^ 3)
        outv[lax.shift_right_logical(g, 5), pl.ds((g & 31) * L, L)] = (
            lax.shift_right_logical(wordsv[s], b) & 1)
        return carry

    for it in range(N_CHUNKS):
        row0 = wid * ROWS_PER_W + it * ROWS_PER_CHUNK
        pltpu.sync_copy(pos_hbm.at[pl.ds(row0, ROWS_PER_CHUNK)], posv)
        lax.fori_loop(0, P // L, compute_indices, 0)
        pltpu.async_copy(tbl_sp.at[widxv], wordsv, sem).wait()
        lax.fori_loop(0, P // L, extract_bits, 0)
        pltpu.sync_copy(outv, out_hbm.at[pl.ds(row0, ROWS_PER_CHUNK)])


def kernel(positions, is_training, voxel_grid):
    out = _sc_prune(positions, voxel_grid.astype(jnp.uint8))
    return out.astype(jnp.bool_)


# speedup vs baseline: 12.4337x; 10.9958x over previous
import functools

import jax
import jax.numpy as jnp
from jax import lax
from jax.experimental import pallas as pl
from jax.experimental.pallas import tpu as pltpu
from jax.experimental.pallas import tpu_sc as plsc

G = 256
NROWS, NCOLS = 4096, 512
NC, NS, L = 2, 16, 16
NW = NC * NS
ROWS_PER_W = NROWS // NW
ROWS_PER_CHUNK = 8
P = ROWS_PER_CHUNK * NCOLS
N_CHUNKS = ROWS_PER_W // ROWS_PER_CHUNK
PLANES_PER_TILE = G // NS
WORDS_PER_PLANE = G * G // 32


@functools.partial(
    pl.kernel,
    out_type=jax.ShapeDtypeStruct((NROWS, NCOLS), jnp.int32),
    mesh=plsc.VectorSubcoreMesh(core_axis_name="c", subcore_axis_name="s"),
    compiler_params=pltpu.CompilerParams(
        needs_layout_passes=False, use_tc_tiling_on_sc=False
    ),
    scratch_types=[
        pltpu.VMEM_SHARED((G * G * G // 32,), jnp.int32),
        pltpu.VMEM((G, G), jnp.uint8),
        pltpu.VMEM((WORDS_PER_PLANE,), jnp.int32),
        pltpu.VMEM((ROWS_PER_CHUNK, NCOLS), jnp.float32),
        pltpu.VMEM((ROWS_PER_CHUNK, NCOLS), jnp.float32),
        pltpu.VMEM((ROWS_PER_CHUNK, NCOLS), jnp.float32),
        pltpu.VMEM((P,), jnp.int32),
        pltpu.VMEM((P,), jnp.int32),
        pltpu.VMEM((P,), jnp.int32),
        pltpu.VMEM((ROWS_PER_CHUNK, NCOLS), jnp.int32),
        pltpu.SemaphoreType.DMA,
    ],
)
def _sc_prune(xyz_hbm, grid_hbm, out_hbm, tbl_sp, planev, wordbuf, xv, yv, zv,
              linv, widxv, wordsv, outv, sem):
    cid = lax.axis_index("c")
    sid = lax.axis_index("s")
    wid = sid * NC + cid

    def pack_vec(ov, carry):
        r = ov * 2
        w = None
        for j in range(8):
            v8 = planev[r + (j >> 2), pl.ds((j & 3) * 64, 64)]
            v = plsc.bitcast(v8, jnp.int32)
            nib = lax.shift_right_logical(v * 0x08040201, 24)
            w = nib if j == 0 else w | (nib << (4 * j))
        wordbuf[pl.ds(ov * L, L)] = w
        return carry

    for pi in range(PLANES_PER_TILE):
        p = sid * PLANES_PER_TILE + pi
        pltpu.sync_copy(grid_hbm.at[p], planev)
        lax.fori_loop(0, WORDS_PER_PLANE // L, pack_vec, 0)
        pltpu.sync_copy(wordbuf, tbl_sp.at[pl.ds(p * WORDS_PER_PLANE,
                                                 WORDS_PER_PLANE)])
    plsc.subcore_barrier()

    def compute_indices(g, carry):
        r = lax.shift_right_logical(g, 5)
        c = pl.ds((g & 31) * L, L)
        ix = (xv[r, c] * float(G)).astype(jnp.int32)
        iy = (yv[r, c] * float(G)).astype(jnp.int32)
        iz = (zv[r, c] * float(G)).astype(jnp.int32)
        lin = (ix << 16) | (iy << 8) | iz
        s = pl.ds(g * L, L)
        linv[s] = lin
        widxv[s] = (lax.shift_right_logical(lin, 5) & -16) | (
            lax.shift_right_logical(lin, 2) & 15)
        return carry

    def extract_bits(g, carry):
        s = pl.ds(g * L, L)
        lin = linv[s]
        b = (lax.shift_right_logical(lin, 4) & 28) | ((lin & 3) ^ 3)
        outv[lax.shift_right_logical(g, 5), pl.ds((g & 31) * L, L)] = (
            lax.shift_right_logical(wordsv[s], b) & 1)
        return carry

    for it in range(N_CHUNKS):
        row0 = wid * ROWS_PER_W + it * ROWS_PER_CHUNK
        pltpu.sync_copy(xyz_hbm.at[0, pl.ds(row0, ROWS_PER_CHUNK)], xv)
        pltpu.sync_copy(xyz_hbm.at[1, pl.ds(row0, ROWS_PER_CHUNK)], yv)
        pltpu.sync_copy(xyz_hbm.at[2, pl.ds(row0, ROWS_PER_CHUNK)], zv)
        lax.fori_loop(0, P // L, compute_indices, 0)
        pltpu.async_copy(tbl_sp.at[widxv], wordsv, sem).wait()
        lax.fori_loop(0, P // L, extract_bits, 0)
        pltpu.sync_copy(outv, out_hbm.at[pl.ds(row0, ROWS_PER_CHUNK)])


def kernel(positions, is_training, voxel_grid):
    xyz = jnp.moveaxis(positions, 2, 0)
    out = _sc_prune(xyz, voxel_grid.astype(jnp.uint8))
    return out.astype(jnp.bool_)


# speedup vs baseline: 18.3406x; 1.4751x over previous
import functools

import jax
import jax.numpy as jnp
from jax import lax
from jax.experimental import pallas as pl
from jax.experimental.pallas import tpu as pltpu
from jax.experimental.pallas import tpu_sc as plsc

G = 256
NROWS, NCOLS = 4096, 512
NPTS = NROWS * NCOLS
NC, NS, L = 2, 16, 16
NW = NC * NS
PTS_PER_W = NPTS // NW
P = 4096
N_CHUNKS = PTS_PER_W // P
PLANES_PER_TILE = G // NS
WORDS_PER_PLANE = G * G // 32


@functools.partial(
    pl.kernel,
    out_type=jax.ShapeDtypeStruct((NPTS,), jnp.int32),
    mesh=plsc.VectorSubcoreMesh(core_axis_name="c", subcore_axis_name="s"),
    compiler_params=pltpu.CompilerParams(
        needs_layout_passes=False, use_tc_tiling_on_sc=False
    ),
    scratch_types=[
        pltpu.VMEM_SHARED((G * G * G // 32,), jnp.int32),
        pltpu.VMEM((2, G, G), jnp.uint8),
        pltpu.VMEM((2, WORDS_PER_PLANE,), jnp.int32),
        pltpu.VMEM((2, 3, P), jnp.float32),
        pltpu.VMEM((2, P), jnp.int32),
        pltpu.VMEM((2, P), jnp.int32),
        pltpu.VMEM((2, P), jnp.int32),
        pltpu.VMEM((2, P), jnp.int32),
        pltpu.SemaphoreType.DMA,
        pltpu.SemaphoreType.DMA,
        pltpu.SemaphoreType.DMA,
        pltpu.SemaphoreType.DMA,
    ],
)
def _sc_prune(xyz_hbm, grid_hbm, out_hbm, tbl_sp, planev, wordbuf, posv,
              widxv, bidxv, wordsv, outv, psem, gsem, osem, tsem):
    cid = lax.axis_index("c")
    sid = lax.axis_index("s")
    wid = sid * NC + cid

    def pack_plane(pi, buf):
        p = sid * PLANES_PER_TILE + pi

        @plsc.parallel_loop(0, WORDS_PER_PLANE // L, unroll=4)
        def pack_vec(ov):
            r = ov * 2
            w = None
            for j in range(8):
                v8 = planev[buf, r + (j >> 2), pl.ds((j & 3) * 64, 64)]
                v = plsc.bitcast(v8, jnp.int32)
                nib = lax.shift_right_logical(v * 0x08040201, 24)
                w = nib if j == 0 else w | (nib << (4 * j))
            wordbuf[buf, pl.ds(ov * L, L)] = w

        return pltpu.async_copy(
            wordbuf.at[buf],
            tbl_sp.at[pl.ds(p * WORDS_PER_PLANE, WORDS_PER_PLANE)], tsem)

    def plane_dma(pi, buf):
        return pltpu.async_copy(
            grid_hbm.at[sid * PLANES_PER_TILE + pi], planev.at[buf], psem)

    dma = plane_dma(0, 0)
    tput = None
    for pi in range(PLANES_PER_TILE):
        dma.wait()
        if pi + 1 < PLANES_PER_TILE:
            dma = plane_dma(pi + 1, (pi + 1) & 1)
        if tput is not None:
            tput.wait()
        tput = pack_plane(pi, pi & 1)
    tput.wait()
    plsc.subcore_barrier()

    def pos_dma(k, buf):
        base = wid * PTS_PER_W + k * P
        return [
            pltpu.async_copy(xyz_hbm.at[c, pl.ds(base, P)], posv.at[buf, c],
                             psem)
            for c in range(3)
        ]

    def compute_idx(k, buf):
        @plsc.parallel_loop(0, P // L, unroll=4)
        def _(g):
            s = pl.ds(g * L, L)
            ix = (posv[buf, 0, s] * float(G)).astype(jnp.int32)
            iy = (posv[buf, 1, s] * float(G)).astype(jnp.int32)
            iz = (posv[buf, 2, s] * float(G)).astype(jnp.int32)
            lin = (ix << 16) | (iy << 8) | iz
            widxv[buf, s] = (lax.shift_right_logical(lin, 5) & -16) | (
                lax.shift_right_logical(lin, 2) & 15)
            bidxv[buf, s] = (lax.shift_right_logical(lin, 4) & 28) | (
                (lin & 3) ^ 3)

    def extract_bits(k, buf):
        @plsc.parallel_loop(0, P // L, unroll=4)
        def _(g):
            s = pl.ds(g * L, L)
            outv[buf, s] = (
                lax.shift_right_logical(wordsv[buf, s], bidxv[buf, s]) & 1)
        return pltpu.async_copy(
            outv.at[buf], out_hbm.at[pl.ds(wid * PTS_PER_W + k * P, P)], osem)

    pdmas = pos_dma(0, 0)
    gdma = None
    odmas = [None, None]
    for k in range(N_CHUNKS):
        b = k & 1
        for d in pdmas:
            d.wait()
        if k + 1 < N_CHUNKS:
            pdmas = pos_dma(k + 1, 1 - b)
        compute_idx(k, b)
        if gdma is not None:
            gdma.wait()
            if odmas[b] is not None:
                odmas[b].wait()
            odmas[1 - b] = extract_bits(k - 1, 1 - b)
        gdma = pltpu.async_copy(tbl_sp.at[widxv.at[b]], wordsv.at[b], gsem)
    gdma.wait()
    b = (N_CHUNKS - 1) & 1
    last = extract_bits(N_CHUNKS - 1, b)
    odmas[1 - b].wait()
    last.wait()


def kernel(positions, is_training, voxel_grid):
    xyz = jnp.moveaxis(positions, 2, 0).reshape(3, NPTS)
    out = _sc_prune(xyz, voxel_grid.astype(jnp.uint8))
    return out.reshape(NROWS, NCOLS).astype(jnp.bool_)
